# Initial kernel scaffold; baseline (speedup 1.0000x reference)
import jax
import jax.numpy as jnp
from jax.experimental import pallas as pl


def kernel(input, grid):
    raise NotImplementedError("write your pallas kernel here")


# speedup vs baseline: 4.9267x; 4.9267x over previous
import functools

import jax
import jax.numpy as jnp
from jax import lax
from jax.experimental import pallas as pl
from jax.experimental.pallas import tpu as pltpu
from jax.experimental.pallas import tpu_sc as plsc

N, C, H, W = 4, 32, 384, 384
HW = H * W
NP = N * HW
NWORK = 32
PW = NP // NWORK
B = 512
NCHUNK = PW // B
L = 16


def _reflect_floor(v, size):
    span = float(size - 1)
    a = jnp.abs(v)
    extra = jnp.mod(a, span)
    flips = (a / span).astype(jnp.int32)
    r = jnp.where((flips & 1) == 0, extra, span - extra)
    r = jnp.clip(r, 0.0, span)
    i0 = r.astype(jnp.int32)
    frac = r - i0.astype(jnp.float32)
    return i0, frac


def _sc_body(table, gxh, gyh, outh,
             gx_v, gy_v, i00, i01, i10, i11, w00s, w01s, w10s, w11s,
             r00, r01, r10, r11, out_v, sem):
    wid = lax.axis_index("s") * 2 + lax.axis_index("c")
    base_row = (wid // 8) * HW

    def chunk_body(t, carry):
        p0 = wid * PW + t * B
        pltpu.sync_copy(gxh.at[pl.ds(p0, B)], gx_v)
        pltpu.sync_copy(gyh.at[pl.ds(p0, B)], gy_v)

        def idx_body(g, carry2):
            sl = pl.ds(g * L, L)
            ix = (gx_v[sl] + 1.0) * 0.5 * (W - 1)
            iy = (gy_v[sl] + 1.0) * 0.5 * (H - 1)
            ix0, wx1 = _reflect_floor(ix, W)
            iy0, wy1 = _reflect_floor(iy, H)
            ix0c = jnp.minimum(ix0, W - 1)
            ix1c = jnp.minimum(ix0 + 1, W - 1)
            iy0c = jnp.minimum(iy0, H - 1)
            iy1c = jnp.minimum(iy0 + 1, H - 1)
            r0 = base_row + iy0c * W
            r1 = base_row + iy1c * W
            i00[sl] = r0 + ix0c
            i01[sl] = r0 + ix1c
            i10[sl] = r1 + ix0c
            i11[sl] = r1 + ix1c
            wx0 = 1.0 - wx1
            wy0 = 1.0 - wy1
            w00s[sl] = wy0 * wx0
            w01s[sl] = wy0 * wx1
            w10s[sl] = wy1 * wx0
            w11s[sl] = wy1 * wx1
            return carry2

        lax.fori_loop(0, B // L, idx_body, 0)

        cp0 = pltpu.async_copy(table.at[i00], r00, sem)
        cp1 = pltpu.async_copy(table.at[i01], r01, sem)
        cp2 = pltpu.async_copy(table.at[i10], r10, sem)
        cp3 = pltpu.async_copy(table.at[i11], r11, sem)
        cp0.wait()
        cp1.wait()
        cp2.wait()
        cp3.wait()

        def blend_body(g, carry2):
            sl = pl.ds(g * L, L)
            w00v = w00s[sl]
            w01v = w01s[sl]
            w10v = w10s[sl]
            w11v = w11s[sl]
            for i in range(L):
                ci = jnp.full((L,), i, dtype=jnp.int32)
                w00 = w00v.at[ci].get(mode="promise_in_bounds")
                w01 = w01v.at[ci].get(mode="promise_in_bounds")
                w10 = w10v.at[ci].get(mode="promise_in_bounds")
                w11 = w11v.at[ci].get(mode="promise_in_bounds")
                p = g * L + i
                for h in range(C // L):
                    cs = pl.ds(h * L, L)
                    acc = (r00[p, cs] * w00 + r01[p, cs] * w01 +
                           r10[p, cs] * w10 + r11[p, cs] * w11)
                    out_v[p, cs] = acc
            return carry2

        lax.fori_loop(0, B // L, blend_body, 0)
        pltpu.sync_copy(out_v, outh.at[pl.ds(p0, B)])
        return carry

    lax.fori_loop(0, NCHUNK, chunk_body, 0)


@jax.jit
def _grid_sample_sc(tableT, gx, gy):
    mesh = plsc.VectorSubcoreMesh(core_axis_name="c", subcore_axis_name="s")
    kfn = functools.partial(
        pl.kernel,
        mesh=mesh,
        compiler_params=pltpu.CompilerParams(use_tc_tiling_on_sc=False),
        out_type=jax.ShapeDtypeStruct((NP, C), jnp.float32),
        scratch_types=[
            pltpu.VMEM((B,), jnp.float32),
            pltpu.VMEM((B,), jnp.float32),
            pltpu.VMEM((B,), jnp.int32),
            pltpu.VMEM((B,), jnp.int32),
            pltpu.VMEM((B,), jnp.int32),
            pltpu.VMEM((B,), jnp.int32),
            pltpu.VMEM((B,), jnp.float32),
            pltpu.VMEM((B,), jnp.float32),
            pltpu.VMEM((B,), jnp.float32),
            pltpu.VMEM((B,), jnp.float32),
            pltpu.VMEM((B, C), jnp.float32),
            pltpu.VMEM((B, C), jnp.float32),
            pltpu.VMEM((B, C), jnp.float32),
            pltpu.VMEM((B, C), jnp.float32),
            pltpu.VMEM((B, C), jnp.float32),
            pltpu.SemaphoreType.DMA,
        ],
    )(_sc_body)
    return kfn(tableT, gx, gy)


def kernel(input, grid):
    tableT = jnp.transpose(input.reshape(N, C, HW), (0, 2, 1)).reshape(NP, C)
    gx = grid[..., 0].reshape(NP)
    gy = grid[..., 1].reshape(NP)
    out = _grid_sample_sc(tableT, gx, gy)
    return jnp.transpose(out.reshape(N, H, W, C), (0, 3, 1, 2))
